# Initial kernel scaffold; baseline (speedup 1.0000x reference)
import jax
import jax.numpy as jnp
from jax.experimental import pallas as pl


def kernel(x, edge_index, W, b):
    raise NotImplementedError("write your pallas kernel here")


# speedup vs baseline: 28.9678x; 28.9678x over previous
import functools

import jax
import jax.numpy as jnp
from jax import lax
from jax.experimental import pallas as pl
from jax.experimental.pallas import tpu as pltpu
from jax.experimental.pallas import tpu_sc as plsc

N_NODES = 10000
N_EDGES = 320000
D = 128

NC = 2
NS = 16
NW = NC * NS

N_PAD = 10240
NODES_PER_TILE = N_PAD // NS

N_PAD_E = 10112
NODES_PER_TILE_E = N_PAD_E // NS
CHUNK = 128
ROWS_PER_TILE = 80
E_PAD = NW * ROWS_PER_TILE * CHUNK

_mesh = plsc.VectorSubcoreMesh(core_axis_name="c", subcore_axis_name="s")


EDGES_PER_TILE = N_EDGES // NW
HVEC = 16


@functools.partial(
    pl.kernel,
    mesh=_mesh,
    compiler_params=pltpu.CompilerParams(needs_layout_passes=False),
    out_type=jax.ShapeDtypeStruct((NC, N_PAD), jnp.float32),
    scratch_types=[
        pltpu.VMEM((EDGES_PER_TILE,), jnp.int32),
        pltpu.VMEM((N_PAD,), jnp.float32),
        pltpu.VMEM((NS, NODES_PER_TILE), jnp.float32),
        pltpu.VMEM_SHARED((NS, N_PAD), jnp.float32),
    ],
)
def _deg_kernel(dst_hbm, zeros_hbm, degp_hbm, idx_v, hist, tbuf, stage):
    c = lax.axis_index("c")
    s = lax.axis_index("s")
    tid = c * NS + s
    pltpu.sync_copy(zeros_hbm, hist)
    pltpu.sync_copy(dst_hbm.at[pl.ds(tid * EDGES_PER_TILE, EDGES_PER_TILE)], idx_v)
    ones16 = jnp.full((HVEC,), 1.0, jnp.float32)

    def body(j, carry):
        v = idx_v[pl.ds(j * HVEC, HVEC)]
        plsc.addupdate_scatter(hist, [v], ones16)
        return carry

    lax.fori_loop(0, EDGES_PER_TILE // HVEC, body, 0)
    pltpu.sync_copy(hist, stage.at[s])
    plsc.subcore_barrier()
    for t in range(NS):
        pltpu.sync_copy(
            stage.at[t, pl.ds(s * NODES_PER_TILE, NODES_PER_TILE)], tbuf.at[t]
        )

    def red(i, carry):
        acc = tbuf[0, pl.ds(i * HVEC, HVEC)]
        for t in range(1, NS):
            acc = acc + tbuf[t, pl.ds(i * HVEC, HVEC)]
        hist[pl.ds(s * NODES_PER_TILE + i * HVEC, HVEC)] = acc
        return carry

    lax.fori_loop(0, NODES_PER_TILE // HVEC, red, 0)
    pltpu.sync_copy(
        hist.at[pl.ds(s * NODES_PER_TILE, NODES_PER_TILE)],
        degp_hbm.at[c, pl.ds(s * NODES_PER_TILE, NODES_PER_TILE)],
    )


@functools.partial(
    pl.kernel,
    mesh=_mesh,
    out_type=jax.ShapeDtypeStruct((NC, N_PAD_E, D), jnp.float32),
    scratch_types=[
        pltpu.VMEM((ROWS_PER_TILE, CHUNK), jnp.int32),
        pltpu.VMEM((ROWS_PER_TILE, CHUNK), jnp.int32),
        pltpu.VMEM((CHUNK, D), jnp.float32),
        pltpu.VMEM_SHARED((N_PAD_E, D), jnp.float32),
        pltpu.SemaphoreType.DMA,
    ],
)
def _edge_kernel(g_hbm, src_hbm, dst_hbm, acc_hbm, src_v, dst_v, rbuf, accs, sem):
    c = lax.axis_index("c")
    s = lax.axis_index("s")
    tid = c * NS + s

    r0 = s * NODES_PER_TILE_E
    for k in range(4):
        rk = r0 + k * CHUNK
        pltpu.sync_copy(g_hbm.at[pl.ds(rk, CHUNK)], rbuf)
        pltpu.sync_copy(rbuf, accs.at[pl.ds(rk, CHUNK)])
    rk = r0 + 4 * CHUNK
    pltpu.sync_copy(g_hbm.at[pl.ds(rk, 120)], rbuf.at[pl.ds(0, 120)])
    pltpu.sync_copy(rbuf.at[pl.ds(0, 120)], accs.at[pl.ds(rk, 120)])
    pltpu.sync_copy(src_hbm.at[tid], src_v)
    pltpu.sync_copy(dst_hbm.at[tid], dst_v)
    plsc.subcore_barrier()

    def body(j, carry):
        pltpu.async_copy(g_hbm.at[src_v.at[j]], rbuf, sem).wait()
        pltpu.sync_copy(rbuf, accs.at[dst_v.at[j]], add=True)
        return carry

    lax.fori_loop(0, ROWS_PER_TILE, body, 0)
    plsc.subcore_barrier()
    for k in range(4):
        rk = r0 + k * CHUNK
        pltpu.sync_copy(accs.at[pl.ds(rk, CHUNK)], rbuf)
        pltpu.sync_copy(rbuf, acc_hbm.at[c, pl.ds(rk, CHUNK)])
    rk = r0 + 4 * CHUNK
    pltpu.sync_copy(accs.at[pl.ds(rk, 120)], rbuf.at[pl.ds(0, 120)])
    pltpu.sync_copy(rbuf.at[pl.ds(0, 120)], acc_hbm.at[c, pl.ds(rk, 120)])


_BLK = 1000


def _transform_body(x_ref, w_ref, dp_ref, g_ref, dv_ref):
    h = jnp.dot(x_ref[...], w_ref[...], preferred_element_type=jnp.float32)
    deg = dp_ref[0] + dp_ref[1] + 1.0
    dinv = lax.rsqrt(jnp.maximum(deg, 1.0))
    g_ref[...] = h * dinv
    dv_ref[...] = jnp.broadcast_to(dinv, h.shape)


def _transform(x, W, degp):
    grid = (N_NODES // _BLK,)
    return pl.pallas_call(
        _transform_body,
        grid=grid,
        in_specs=[
            pl.BlockSpec((_BLK, D), lambda i: (i, 0)),
            pl.BlockSpec((D, D), lambda i: (0, 0)),
            pl.BlockSpec((NC, _BLK, 1), lambda i: (0, i, 0)),
        ],
        out_specs=[
            pl.BlockSpec((_BLK, D), lambda i: (i, 0)),
            pl.BlockSpec((_BLK, D), lambda i: (i, 0)),
        ],
        out_shape=[
            jax.ShapeDtypeStruct((N_NODES, D), jnp.float32),
            jax.ShapeDtypeStruct((N_NODES, D), jnp.float32),
        ],
    )(x, W, degp)


def _final_body(acc_ref, g_ref, dv_ref, b_ref, o_ref):
    t = acc_ref[0] + acc_ref[1] - g_ref[...]
    o_ref[...] = jnp.maximum(dv_ref[...] * t + b_ref[...], 0.0)


def _finalize(acc, g, dv, b2):
    grid = (N_NODES // _BLK,)
    return pl.pallas_call(
        _final_body,
        grid=grid,
        in_specs=[
            pl.BlockSpec((NC, _BLK, D), lambda i: (0, i, 0)),
            pl.BlockSpec((_BLK, D), lambda i: (i, 0)),
            pl.BlockSpec((_BLK, D), lambda i: (i, 0)),
            pl.BlockSpec((1, D), lambda i: (0, 0)),
        ],
        out_specs=pl.BlockSpec((_BLK, D), lambda i: (i, 0)),
        out_shape=jax.ShapeDtypeStruct((N_NODES, D), jnp.float32),
    )(acc, g, dv, b2)


def kernel(x, edge_index, W, b):
    ei = edge_index.astype(jnp.int32)
    pad_idx = N_NODES + jnp.arange(E_PAD - N_EDGES, dtype=jnp.int32) % (
        N_PAD_E - N_NODES)
    ei_pad = jnp.concatenate([ei, jnp.stack([pad_idx, pad_idx])], axis=1)
    src = ei_pad[0].reshape(NW, ROWS_PER_TILE, CHUNK)
    dst = ei_pad[1].reshape(NW, ROWS_PER_TILE, CHUNK)
    dst_flat = ei[1]
    zeros_h = jnp.zeros((N_PAD,), jnp.float32)
    degp = _deg_kernel(dst_flat, zeros_h)
    g, dv = _transform(x, W, degp[:, :N_NODES, None])
    g_pad = jnp.concatenate(
        [g, jnp.zeros((N_PAD_E - N_NODES, D), jnp.float32)], axis=0
    )
    acc = _edge_kernel(g_pad, src, dst)
    return _finalize(acc[:, :N_NODES], g, dv, b.reshape(1, D))
